# baseline (speedup 1.0000x reference)
import functools

import jax
import jax.numpy as jnp
from jax import lax
from jax.experimental import pallas as pl
from jax.experimental.pallas import tpu as pltpu
from jax.experimental.pallas import tpu_sc as plsc

N = 10000
E = 320000
H = 128

NC = 2
NS = 16
L = 16

EPS_SC = NC * NS
EPT = E // EPS_SC
K = 80
NCHUNK = EPT // K
N_PAD = 10112
ROWS_PER_TILE = N_PAD // NS


def _edge_pass_body(mul, idx_hbm, val_hbm, x_hbm, zeros_hbm,
                    part_hbm, acc, ii0, ii1, ii2,
                    xg0, xg1, val0, val1,
                    sem_i0, sem_i1, sem_i2,
                    sem_v0, sem_v1, sem_g0, sem_g1, sem_s0, sem_s1):
    c = lax.axis_index("c")
    s = lax.axis_index("s")
    wid = c * NS + s
    tile_base = wid * EPT
    ii = (ii0, ii1, ii2)
    xg = (xg0, xg1)
    val = (val0, val1)
    sem_i = (sem_i0, sem_i1, sem_i2)
    sem_v = (sem_v0, sem_v1)
    sem_g = (sem_g0, sem_g1)
    sem_s = (sem_s0, sem_s1)

    pltpu.sync_copy(zeros_hbm,
                    acc.at[pl.ds(s * ROWS_PER_TILE, ROWS_PER_TILE), :])
    plsc.subcore_barrier()

    def _issue_idx(g, t):
        pltpu.async_copy(idx_hbm.at[wid, g], ii[t], sem_i[t])

    def _issue_val(g, b):
        pltpu.async_copy(val_hbm.at[pl.ds(tile_base + g * K, K), :],
                         val[b], sem_v[b])

    def _issue_gather(t, b):
        pltpu.async_copy(x_hbm.at[ii[t].at[0]], xg[b], sem_g[b])

    def _wait_si(t):
        pltpu.make_async_copy(idx_hbm.at[0, 0], ii[t], sem_i[t]).wait()

    def _wait_di(t):
        pass

    def _wait_val(b):
        pltpu.make_async_copy(val_hbm.at[pl.ds(0, K), :], val[b],
                              sem_v[b]).wait()

    def _wait_gather(b):
        pltpu.make_async_copy(x_hbm.at[ii[0].at[0]], xg[b], sem_g[b]).wait()

    def _wait_scat(b):
        pltpu.make_async_copy(xg[b], acc.at[ii[0].at[1]], sem_s[b]).wait()

    def _step(g, b, t, first=False, g1_pre_issued=False, gnext=None):
        gn = g + 2 if gnext is None else gnext
        _wait_val(b)
        _wait_gather(b)
        _wait_di(t)

        @plsc.parallel_loop(0, K, step=1, unroll=2)
        def _mrow(r):
            for j in range(H // L):
                sl = pl.ds(j * L, L)
                if mul:
                    xg[b][r, sl] = val[b][r, sl] * xg[b][r, sl]
                else:
                    xg[b][r, sl] = val[b][r, sl] + xg[b][r, sl]
        _issue_val(gn, b)
        pltpu.async_copy(xg[b], acc.at[ii[t].at[1]], sem_s[b], add=True)
        if not first:
            _wait_scat(1 - b)
        if not g1_pre_issued:
            _wait_si((t + 1) % 3)
            _issue_gather((t + 1) % 3, 1 - b)
        if not first:
            _issue_idx(gn, (t + 2) % 3)

    _issue_idx(0, 0)
    _issue_idx(1, 1)
    _issue_idx(2, 2)
    _issue_val(0, 0)
    _issue_val(1, 1)
    _wait_si(0)
    _issue_gather(0, 0)
    _wait_si(1)
    _issue_gather(1, 1)

    _step(0, 0, 0, first=True, g1_pre_issued=True)

    def _six(i, carry):
        g = 1 + 6 * i
        for u in range(6):
            _step(g + u, (1 + u) % 2, (1 + u) % 3)
        return carry
    lax.fori_loop(0, 20, _six, 0)

    for g in range(121, NCHUNK):
        _step(g, g % 2, g % 3, gnext=min(g + 2, NCHUNK - 1))

    _wait_si(0)
    _wait_val(1)
    _wait_val(0)
    _wait_gather(1)
    _wait_scat(0)

    plsc.subcore_barrier()
    pltpu.sync_copy(acc.at[pl.ds(s * ROWS_PER_TILE, ROWS_PER_TILE), :],
                    part_hbm.at[c, pl.ds(s * ROWS_PER_TILE, ROWS_PER_TILE), :])


def _make_edge_pass(mul):
    mesh = plsc.VectorSubcoreMesh(core_axis_name="c", subcore_axis_name="s")
    return functools.partial(
        pl.kernel,
        functools.partial(_edge_pass_body, mul),
        mesh=mesh,
        out_type=jax.ShapeDtypeStruct((NC, N_PAD, H), jnp.float32),
        scratch_types=(
            [pltpu.VMEM_SHARED((N_PAD, H), jnp.float32)]
            + [pltpu.VMEM((2, K), jnp.int32) for _ in range(3)]
            + [pltpu.VMEM((K, H), jnp.float32) for _ in range(4)]
            + [pltpu.SemaphoreType.DMA for _ in range(9)]
        ),
    )()


def _zeros_rows():
    return jnp.zeros((ROWS_PER_TILE, H), jnp.float32)


def _pack_idx(src, dst):
    return jnp.stack([src.reshape(EPS_SC, NCHUNK, K),
                      dst.reshape(EPS_SC, NCHUNK, K)], axis=2)


def _edge_pass_mul(ps, pd, val, x):
    return _make_edge_pass(True)(_pack_idx(ps, pd), val, x, _zeros_rows())


def _edge_pass_add(es, ed, val, x):
    return _make_edge_pass(False)(_pack_idx(es, ed), val, x, _zeros_rows())


def _edge_mlp_body(f1t_ref, f2t_ref, W1a, b1a, W1b, b1b, W2a, b2a, W2b, b2b,
                   o1_ref, o2_ref):
    t1 = jnp.maximum(
        jnp.dot(W1a[...], f1t_ref[...], preferred_element_type=jnp.float32)
        + b1a[...], 0.0)
    o1_ref[...] = lax.dot_general(
        t1, W1b[...], (((0,), (1,)), ((), ())),
        preferred_element_type=jnp.float32) + b1b[...]
    t2 = jnp.maximum(
        jnp.dot(W2a[...], f2t_ref[...], preferred_element_type=jnp.float32)
        + b2a[...], 0.0)
    o2_ref[...] = lax.dot_general(
        t2, W2b[...], (((0,), (1,)), ((), ())),
        preferred_element_type=jnp.float32) + b2b[...]


def _edge_mlps(f1t, f2t, W1a, b1a, W1b, b1b, W2a, b2a, W2b, b2b):
    BE = 16000
    grid = (E // BE,)
    F1 = f1t.shape[0]
    F2 = f2t.shape[0]
    MID = W1a.shape[0]
    full = lambda shape: pl.BlockSpec(shape, lambda i: (0,) * len(shape))
    return pl.pallas_call(
        _edge_mlp_body,
        grid=grid,
        in_specs=[
            pl.BlockSpec((F1, BE), lambda i: (0, i)),
            pl.BlockSpec((F2, BE), lambda i: (0, i)),
            full((MID, F1)), full((MID, 1)), full((H, MID)), full((1, H)),
            full((MID, F2)), full((MID, 1)), full((H, MID)), full((1, H)),
        ],
        out_specs=[
            pl.BlockSpec((BE, H), lambda i: (i, 0)),
            pl.BlockSpec((BE, H), lambda i: (i, 0)),
        ],
        out_shape=[
            jax.ShapeDtypeStruct((E, H), jnp.float32),
            jax.ShapeDtypeStruct((E, H), jnp.float32),
        ],
    )(f1t, f2t, W1a, b1a, W1b, b1b, W2a, b2a, W2b, b2b)


def _node_body(x_ref, p1_ref, p2_ref, p3_ref,
               Wrel1T, brel1, Wroot1T, Wrel2T, brel2, Wroot2T,
               WcatT, bcat, Wg1T, bg1, ln_g, ln_b, Wg2T, bg2, eps_ref,
               out_ref):
    x = x_ref[...]
    dot = lambda a, b: jnp.dot(a, b, preferred_element_type=jnp.float32)
    agg1 = p1_ref[0] + p1_ref[1]
    h1 = jnp.maximum(dot(agg1, Wrel1T[...]) + brel1[...]
                     + dot(x, Wroot1T[...]), 0.0)
    agg2 = p2_ref[0] + p2_ref[1]
    h2 = jnp.maximum(dot(agg2, Wrel2T[...]) + brel2[...]
                     + dot(x, Wroot2T[...]), 0.0)
    hc = jnp.maximum(dot(h1, WcatT[0:H, :]) + dot(h2, WcatT[H:2 * H, :])
                     + bcat[...], 0.0)
    agg3 = p3_ref[0] + p3_ref[1]
    pre = (1.0 + eps_ref[0, 0]) * x + agg3
    t = dot(pre, Wg1T[...]) + bg1[...]
    mu = jnp.mean(t, axis=-1, keepdims=True)
    var = jnp.mean(jnp.square(t - mu), axis=-1, keepdims=True)
    t = (t - mu) * jax.lax.rsqrt(var + 1e-5) * ln_g[...] + ln_b[...]
    t = jnp.maximum(t, 0.0)
    out_ref[...] = hc + dot(t, Wg2T[...]) + bg2[...]


def _node_stage(x, p1, p2, p3, Wrel1T, brel1, Wroot1T, Wrel2T, brel2, Wroot2T,
                WcatT, bcat, Wg1T, bg1, ln_g, ln_b, Wg2T, bg2, eps):
    BN = 2000
    grid = (N // BN,)
    full = lambda shape: pl.BlockSpec(shape, lambda i: (0,) * len(shape))
    part = pl.BlockSpec((NC, BN, H), lambda i: (0, i, 0))
    return pl.pallas_call(
        _node_body,
        grid=grid,
        in_specs=[
            pl.BlockSpec((BN, H), lambda i: (i, 0)),
            part, part, part,
            full((H, H)), full((1, H)), full((H, H)),
            full((H, H)), full((1, H)), full((H, H)),
            full((2 * H, H)), full((1, H)),
            full((H, H)), full((1, H)), full((1, H)), full((1, H)),
            full((H, H)), full((1, H)), full((1, 1)),
        ],
        out_specs=pl.BlockSpec((BN, H), lambda i: (i, 0)),
        out_shape=jax.ShapeDtypeStruct((N, H), jnp.float32),
    )(x, p1, p2, p3, Wrel1T, brel1, Wroot1T, Wrel2T, brel2, Wroot2T,
      WcatT, bcat, Wg1T, bg1, ln_g, ln_b, Wg2T, bg2, eps)


def kernel(x, feature1, feature2, pos_edge_index, edge_index, edge_attr,
           W1a, b1a, W1b, b1b, W2a, b2a, W2b, b2b,
           Wrel1, brel1, Wroot1, Wrel2, brel2, Wroot2,
           Wcat, bcat, Wg1, bg1, ln_g, ln_b, Wg2, bg2, eps):
    ps = pos_edge_index[0].astype(jnp.int32)
    pd = pos_edge_index[1].astype(jnp.int32)
    es = edge_index[0].astype(jnp.int32)
    ed = edge_index[1].astype(jnp.int32)

    row = lambda b: b.reshape(1, -1)
    col = lambda b: b.reshape(-1, 1)
    f1, f2 = _edge_mlps(feature1.T, feature2.T,
                        W1a, col(b1a), W1b, row(b1b),
                        W2a, col(b2a), W2b, row(b2b))

    part1 = _edge_pass_mul(ps, pd, f1, x)
    part2 = _edge_pass_mul(ps, pd, f2, x)
    part3 = _edge_pass_add(es, ed, edge_attr, x)

    return _node_stage(x, part1, part2, part3,
                       Wrel1.T, row(brel1), Wroot1.T,
                       Wrel2.T, row(brel2), Wroot2.T,
                       Wcat.T, row(bcat), Wg1.T, row(bg1),
                       row(ln_g), row(ln_b), Wg2.T, row(bg2),
                       eps.reshape(1, 1))


# speedup vs baseline: 1.0139x; 1.0139x over previous
import functools

import jax
import jax.numpy as jnp
from jax import lax
from jax.experimental import pallas as pl
from jax.experimental.pallas import tpu as pltpu
from jax.experimental.pallas import tpu_sc as plsc

N = 10000
E = 320000
H = 128

NC = 2
NS = 16
L = 16

EPS_SC = NC * NS
EPT = E // EPS_SC
K = 80
NCHUNK = EPT // K
N_PAD = 10112
ROWS_PER_TILE = N_PAD // NS


def _edge_pass_body(mul, src_hbm, dst_hbm, val_hbm, x_hbm, zeros_hbm,
                    part_hbm, acc, si0, si1, si2, di0, di1, di2,
                    xg0, xg1, val0, val1,
                    sem_si0, sem_si1, sem_si2, sem_di0, sem_di1, sem_di2,
                    sem_v0, sem_v1, sem_g0, sem_g1, sem_s0, sem_s1):
    c = lax.axis_index("c")
    s = lax.axis_index("s")
    wid = c * NS + s
    tile_base = wid * EPT
    si = (si0, si1, si2)
    di = (di0, di1, di2)
    xg = (xg0, xg1)
    val = (val0, val1)
    sem_si = (sem_si0, sem_si1, sem_si2)
    sem_di = (sem_di0, sem_di1, sem_di2)
    sem_v = (sem_v0, sem_v1)
    sem_g = (sem_g0, sem_g1)
    sem_s = (sem_s0, sem_s1)

    pltpu.sync_copy(zeros_hbm,
                    acc.at[pl.ds(s * ROWS_PER_TILE, ROWS_PER_TILE), :])
    plsc.subcore_barrier()

    def _issue_idx(g, t):
        base = tile_base + g * K
        pltpu.async_copy(src_hbm.at[pl.ds(base, K)], si[t], sem_si[t])
        pltpu.async_copy(dst_hbm.at[pl.ds(base, K)], di[t], sem_di[t])

    def _issue_val(g, b):
        pltpu.async_copy(val_hbm.at[pl.ds(tile_base + g * K, K), :],
                         val[b], sem_v[b])

    def _issue_gather(t, b):
        pltpu.async_copy(x_hbm.at[si[t]], xg[b], sem_g[b])

    def _wait_si(t):
        pltpu.make_async_copy(src_hbm.at[pl.ds(0, K)], si[t],
                              sem_si[t]).wait()

    def _wait_di(t):
        pltpu.make_async_copy(dst_hbm.at[pl.ds(0, K)], di[t],
                              sem_di[t]).wait()

    def _wait_val(b):
        pltpu.make_async_copy(val_hbm.at[pl.ds(0, K), :], val[b],
                              sem_v[b]).wait()

    def _wait_gather(b):
        pltpu.make_async_copy(x_hbm.at[si[0]], xg[b], sem_g[b]).wait()

    def _wait_scat(b):
        pltpu.make_async_copy(xg[b], acc.at[di[0]], sem_s[b]).wait()

    def _step(g, b, t, first=False, g1_pre_issued=False, gnext=None):
        gn = g + 2 if gnext is None else gnext
        _wait_val(b)
        _wait_gather(b)
        _wait_di(t)

        @plsc.parallel_loop(0, K, step=1, unroll=2)
        def _mrow(r):
            for j in range(H // L):
                sl = pl.ds(j * L, L)
                if mul:
                    xg[b][r, sl] = val[b][r, sl] * xg[b][r, sl]
                else:
                    xg[b][r, sl] = val[b][r, sl] + xg[b][r, sl]
        if not first:
            _wait_scat(1 - b)
        if not g1_pre_issued:
            _wait_si((t + 1) % 3)
            _issue_gather((t + 1) % 3, 1 - b)
        _issue_val(gn, b)
        pltpu.async_copy(xg[b], acc.at[di[t]], sem_s[b], add=True)
        if not first:
            _issue_idx(gn, (t + 2) % 3)

    _issue_idx(0, 0)
    _issue_idx(1, 1)
    _issue_idx(2, 2)
    _issue_val(0, 0)
    _issue_val(1, 1)
    _wait_si(0)
    _issue_gather(0, 0)
    _wait_si(1)
    _issue_gather(1, 1)

    _step(0, 0, 0, first=True, g1_pre_issued=True)

    def _six(i, carry):
        g = 1 + 6 * i
        for u in range(6):
            _step(g + u, (1 + u) % 2, (1 + u) % 3)
        return carry
    lax.fori_loop(0, 20, _six, 0)

    for g in range(121, NCHUNK):
        _step(g, g % 2, g % 3, gnext=min(g + 2, NCHUNK - 1))

    _wait_si(0)
    _wait_di(2)
    _wait_di(0)
    _wait_val(1)
    _wait_val(0)
    _wait_gather(1)
    _wait_scat(0)

    plsc.subcore_barrier()
    pltpu.sync_copy(acc.at[pl.ds(s * ROWS_PER_TILE, ROWS_PER_TILE), :],
                    part_hbm.at[c, pl.ds(s * ROWS_PER_TILE, ROWS_PER_TILE), :])


def _make_edge_pass(mul):
    mesh = plsc.VectorSubcoreMesh(core_axis_name="c", subcore_axis_name="s")
    return functools.partial(
        pl.kernel,
        functools.partial(_edge_pass_body, mul),
        mesh=mesh,
        out_type=jax.ShapeDtypeStruct((NC, N_PAD, H), jnp.float32),
        scratch_types=(
            [pltpu.VMEM_SHARED((N_PAD, H), jnp.float32)]
            + [pltpu.VMEM((K,), jnp.int32) for _ in range(6)]
            + [pltpu.VMEM((K, H), jnp.float32) for _ in range(4)]
            + [pltpu.SemaphoreType.DMA for _ in range(12)]
        ),
    )()


def _zeros_rows():
    return jnp.zeros((ROWS_PER_TILE, H), jnp.float32)


def _edge_pass_mul(ps, pd, val, x):
    return _make_edge_pass(True)(ps, pd, val, x, _zeros_rows())


def _edge_pass_add(es, ed, val, x):
    return _make_edge_pass(False)(es, ed, val, x, _zeros_rows())


def _edge_mlp_body(f1t_ref, f2t_ref, W1a, b1a, W1b, b1b, W2a, b2a, W2b, b2b,
                   o1_ref, o2_ref):
    t1 = jnp.maximum(
        jnp.dot(W1a[...], f1t_ref[...], preferred_element_type=jnp.float32)
        + b1a[...], 0.0)
    o1_ref[...] = lax.dot_general(
        t1, W1b[...], (((0,), (1,)), ((), ())),
        preferred_element_type=jnp.float32) + b1b[...]
    t2 = jnp.maximum(
        jnp.dot(W2a[...], f2t_ref[...], preferred_element_type=jnp.float32)
        + b2a[...], 0.0)
    o2_ref[...] = lax.dot_general(
        t2, W2b[...], (((0,), (1,)), ((), ())),
        preferred_element_type=jnp.float32) + b2b[...]


def _edge_mlps(f1t, f2t, W1a, b1a, W1b, b1b, W2a, b2a, W2b, b2b):
    BE = 16000
    grid = (E // BE,)
    F1 = f1t.shape[0]
    F2 = f2t.shape[0]
    MID = W1a.shape[0]
    full = lambda shape: pl.BlockSpec(shape, lambda i: (0,) * len(shape))
    return pl.pallas_call(
        _edge_mlp_body,
        grid=grid,
        in_specs=[
            pl.BlockSpec((F1, BE), lambda i: (0, i)),
            pl.BlockSpec((F2, BE), lambda i: (0, i)),
            full((MID, F1)), full((MID, 1)), full((H, MID)), full((1, H)),
            full((MID, F2)), full((MID, 1)), full((H, MID)), full((1, H)),
        ],
        out_specs=[
            pl.BlockSpec((BE, H), lambda i: (i, 0)),
            pl.BlockSpec((BE, H), lambda i: (i, 0)),
        ],
        out_shape=[
            jax.ShapeDtypeStruct((E, H), jnp.float32),
            jax.ShapeDtypeStruct((E, H), jnp.float32),
        ],
    )(f1t, f2t, W1a, b1a, W1b, b1b, W2a, b2a, W2b, b2b)


def _node_body(x_ref, p1_ref, p2_ref, p3_ref,
               Wrel1T, brel1, Wroot1T, Wrel2T, brel2, Wroot2T,
               WcatT, bcat, Wg1T, bg1, ln_g, ln_b, Wg2T, bg2, eps_ref,
               out_ref):
    x = x_ref[...]
    dot = lambda a, b: jnp.dot(a, b, preferred_element_type=jnp.float32)
    agg1 = p1_ref[0] + p1_ref[1]
    h1 = jnp.maximum(dot(agg1, Wrel1T[...]) + brel1[...]
                     + dot(x, Wroot1T[...]), 0.0)
    agg2 = p2_ref[0] + p2_ref[1]
    h2 = jnp.maximum(dot(agg2, Wrel2T[...]) + brel2[...]
                     + dot(x, Wroot2T[...]), 0.0)
    hc = jnp.maximum(dot(h1, WcatT[0:H, :]) + dot(h2, WcatT[H:2 * H, :])
                     + bcat[...], 0.0)
    agg3 = p3_ref[0] + p3_ref[1]
    pre = (1.0 + eps_ref[0, 0]) * x + agg3
    t = dot(pre, Wg1T[...]) + bg1[...]
    mu = jnp.mean(t, axis=-1, keepdims=True)
    var = jnp.mean(jnp.square(t - mu), axis=-1, keepdims=True)
    t = (t - mu) * jax.lax.rsqrt(var + 1e-5) * ln_g[...] + ln_b[...]
    t = jnp.maximum(t, 0.0)
    out_ref[...] = hc + dot(t, Wg2T[...]) + bg2[...]


def _node_stage(x, p1, p2, p3, Wrel1T, brel1, Wroot1T, Wrel2T, brel2, Wroot2T,
                WcatT, bcat, Wg1T, bg1, ln_g, ln_b, Wg2T, bg2, eps):
    BN = 2000
    grid = (N // BN,)
    full = lambda shape: pl.BlockSpec(shape, lambda i: (0,) * len(shape))
    part = pl.BlockSpec((NC, BN, H), lambda i: (0, i, 0))
    return pl.pallas_call(
        _node_body,
        grid=grid,
        in_specs=[
            pl.BlockSpec((BN, H), lambda i: (i, 0)),
            part, part, part,
            full((H, H)), full((1, H)), full((H, H)),
            full((H, H)), full((1, H)), full((H, H)),
            full((2 * H, H)), full((1, H)),
            full((H, H)), full((1, H)), full((1, H)), full((1, H)),
            full((H, H)), full((1, H)), full((1, 1)),
        ],
        out_specs=pl.BlockSpec((BN, H), lambda i: (i, 0)),
        out_shape=jax.ShapeDtypeStruct((N, H), jnp.float32),
    )(x, p1, p2, p3, Wrel1T, brel1, Wroot1T, Wrel2T, brel2, Wroot2T,
      WcatT, bcat, Wg1T, bg1, ln_g, ln_b, Wg2T, bg2, eps)


def kernel(x, feature1, feature2, pos_edge_index, edge_index, edge_attr,
           W1a, b1a, W1b, b1b, W2a, b2a, W2b, b2b,
           Wrel1, brel1, Wroot1, Wrel2, brel2, Wroot2,
           Wcat, bcat, Wg1, bg1, ln_g, ln_b, Wg2, bg2, eps):
    ps = pos_edge_index[0].astype(jnp.int32)
    pd = pos_edge_index[1].astype(jnp.int32)
    es = edge_index[0].astype(jnp.int32)
    ed = edge_index[1].astype(jnp.int32)

    row = lambda b: b.reshape(1, -1)
    col = lambda b: b.reshape(-1, 1)
    f1, f2 = _edge_mlps(feature1.T, feature2.T,
                        W1a, col(b1a), W1b, row(b1b),
                        W2a, col(b2a), W2b, row(b2b))

    part1 = _edge_pass_mul(ps, pd, f1, x)
    part2 = _edge_pass_mul(ps, pd, f2, x)
    part3 = _edge_pass_add(es, ed, edge_attr, x)

    return _node_stage(x, part1, part2, part3,
                       Wrel1.T, row(brel1), Wroot1.T,
                       Wrel2.T, row(brel2), Wroot2.T,
                       Wcat.T, row(bcat), Wg1.T, row(bg1),
                       row(ln_g), row(ln_b), Wg2.T, row(bg2),
                       eps.reshape(1, 1))
